# Initial kernel scaffold; baseline (speedup 1.0000x reference)
import jax
import jax.numpy as jnp
from jax.experimental import pallas as pl


def kernel(x, y, beta_in):
    raise NotImplementedError("write your pallas kernel here")


# speedup vs baseline: 1.0271x; 1.0271x over previous
import functools

import numpy as np
import jax
import jax.numpy as jnp
from jax.experimental import pallas as pl
from jax.experimental.pallas import tpu as pltpu

_K = 5
_MARGIN = 0.2
_CUTOFF = 0.5
_NZCUT = 1.4


def _row_block(n, cap):
    best = 8
    for b in range(8, cap + 1, 8):
        if n % b == 0:
            best = b
    return best


def _logw_block(xi, xa, row0):
    BR, d = xi.shape
    n = xa.shape[0]
    G = jax.lax.dot_general(xi, xa, (((1,), (1,)), ((), ())),
                            preferred_element_type=jnp.float32)
    sqi = jnp.sum(xi * xi, axis=1, keepdims=True)
    ones = jnp.ones((8, d), jnp.float32)
    sqa = jax.lax.dot_general(ones, xa * xa, (((1,), (1,)), ((), ())),
                              preferred_element_type=jnp.float32)[0:1]
    rows = row0 + jax.lax.broadcasted_iota(jnp.int32, (BR, n), 0)
    cols = jax.lax.broadcasted_iota(jnp.int32, (BR, n), 1)
    dist2 = sqi + sqa - 2.0 * G + jnp.where(rows == cols, 1.0, 0.0)
    dis = jnp.sqrt(jnp.maximum(dist2, 1e-12))
    dis = jnp.maximum(dis, _CUTOFF)
    log_w = ((2.0 - float(d)) * jnp.log(dis)
             - (float(d - 3) / 2.0) * jnp.log(jnp.maximum(1.0 - 0.25 * dis * dis, 1e-8)))
    keep = jnp.logical_and(rows // _K != cols // _K, dis < _NZCUT)
    return log_w, keep


def _stats_body(xi_ref, xa_ref, m_ref, s_ref, raw_ref):
    i = pl.program_id(0)
    xi = xi_ref[...]
    xa = xa_ref[...]
    BR = xi.shape[0]
    log_w, keep = _logw_block(xi, xa, i * BR)
    raw = jnp.max(log_w, axis=1, keepdims=True)
    mker = jnp.where(keep, log_w, -1e30)
    m = jnp.max(mker, axis=1, keepdims=True)
    e = jnp.where(keep, jnp.exp(log_w - m), 0.0)
    s = jnp.sum(e, axis=1, keepdims=True)
    m_ref[...] = jnp.broadcast_to(m, m_ref.shape)
    s_ref[...] = jnp.broadcast_to(s, s_ref.shape)
    raw_ref[...] = jnp.broadcast_to(raw, raw_ref.shape)


def _sample_body(c_ref, mm_ref, xi_ref, xa_ref, g_ref, idx_ref, logits_s):
    i = pl.program_id(0)
    xi = xi_ref[...]
    BR = xi.shape[0]
    n = xa_ref.shape[0]

    @pl.when(pl.program_id(1) == 0)
    def _():
        log_w, keep = _logw_block(xi, xa_ref[...], i * BR)
        w = jnp.where(keep, jnp.exp(log_w - mm_ref[0, 0]), 0.0)
        wn = w / c_ref[...]
        logits_s[...] = jnp.log(wn + 1e-12)

    vals = logits_s[...] + g_ref[0]
    mx = jnp.max(vals, axis=1, keepdims=True)
    cols = jax.lax.broadcasted_iota(jnp.int32, vals.shape, 1)
    idx = jnp.min(jnp.where(vals == mx, cols, n), axis=1, keepdims=True)
    idx_ref[0] = idx


def _sample_negative_indices(xs):
    n, d = xs.shape
    BR1 = _row_block(n, 256)
    m, s, raw = pl.pallas_call(
        _stats_body,
        grid=(n // BR1,),
        in_specs=[
            pl.BlockSpec((BR1, d), lambda i: (i, 0)),
            pl.BlockSpec((n, d), lambda i: (0, 0)),
        ],
        out_specs=[
            pl.BlockSpec((BR1, 128), lambda i: (i, 0)),
            pl.BlockSpec((BR1, 128), lambda i: (i, 0)),
            pl.BlockSpec((BR1, 128), lambda i: (i, 0)),
        ],
        out_shape=[jax.ShapeDtypeStruct((n, 128), jnp.float32)] * 3,
    )(xs, xs)
    mm = jnp.max(raw[:, 0]).reshape(1, 1)
    c = s[:, :1] * jnp.exp(m[:, :1] - mm) + 1e-8

    g = jax.random.gumbel(jax.random.key(42), (_K - 1, n, n), jnp.float32)

    BR2 = _row_block(n, 128)
    idx = pl.pallas_call(
        _sample_body,
        grid=(n // BR2, _K - 1),
        in_specs=[
            pl.BlockSpec((BR2, 1), lambda i, r: (i, 0)),
            pl.BlockSpec((1, 1), lambda i, r: (0, 0)),
            pl.BlockSpec((BR2, d), lambda i, r: (i, 0)),
            pl.BlockSpec((n, d), lambda i, r: (0, 0)),
            pl.BlockSpec((1, BR2, n), lambda i, r: (r, i, 0)),
        ],
        out_specs=pl.BlockSpec((1, BR2, 1), lambda i, r: (r, i, 0)),
        out_shape=jax.ShapeDtypeStruct((_K - 1, n, 1), jnp.int32),
        scratch_shapes=[pltpu.VMEM((BR2, n), jnp.float32)],
    )(c, mm, xs, xs, g)
    return idx[:, :, 0].T.reshape(-1)


def _triplet_indices(n, k):
    a_idx = np.repeat(np.arange(n), k - 1)
    blocks = np.arange(n) // k
    offs = np.arange(k)
    p_full = blocks[:, None] * k + offs[None, :]
    p_keep = p_full != np.arange(n)[:, None]
    p_idx = p_full[p_keep]
    return a_idx, p_idx


def kernel(x, y, beta_in):
    n, d = x.shape
    xs = jax.lax.stop_gradient(x)
    n_index = _sample_negative_indices(xs)

    a_np, p_np = _triplet_indices(n, _K)
    a_idx = jnp.asarray(a_np)
    p_idx = jnp.asarray(p_np)
    beta_work = beta_in[a_idx]
    anchors = x[a_idx]
    positives = x[p_idx]
    negatives = x[n_index]
    d_ap = jnp.sqrt(jnp.sum((anchors - positives) ** 2, axis=1) + 1e-8)
    d_an = jnp.sqrt(jnp.sum((anchors - negatives) ** 2, axis=1) + 1e-8)
    pos_loss = jax.nn.relu(d_ap - beta_work + _MARGIN)
    neg_loss = jax.nn.relu(beta_work - d_an + _MARGIN)
    pair_cnt = jnp.sum(jnp.logical_or(pos_loss > 0.0, neg_loss > 0.0))
    return jnp.sum(pos_loss + neg_loss) / pair_cnt.astype(jnp.float32)
